# baseline (speedup 1.0000x reference)
import functools

import jax
import jax.numpy as jnp
from jax import lax
from jax.experimental import pallas as pl
from jax.experimental.pallas import tpu as pltpu, tpu_sc as plsc

_N_ATOMS = 1000000
_N_GROUND = 500000
_BODY_LEN = 4
_HEAD_LEN = 2
_LANES = 16

_BLK = 128
_NBLK = 3904
_NOCT = _NBLK // 8
_NW = 32
_CBL = 16
_NCH = 8
_BW = _BODY_LEN * _BLK
_HW = _HEAD_LEN * _BLK
_NRULE = 3


def _make_sc_kernel():
    mesh = plsc.VectorSubcoreMesh(core_axis_name="c", subcore_axis_name="s")

    @functools.partial(
        pl.kernel,
        mesh=mesh,
        out_type=jax.ShapeDtypeStruct((_NRULE, _NW, _LANES), jnp.float32),
        compiler_params=pltpu.CompilerParams(
            needs_layout_passes=False,
            use_tc_tiling_on_sc=False,
        ),
        scratch_types=[
            pltpu.VMEM_SHARED((_N_ATOMS,), jnp.float32),
            pltpu.VMEM((_CBL, _BW), jnp.int32),
            pltpu.VMEM((_CBL, _BW), jnp.float32),
            pltpu.VMEM((_CBL, _HW), jnp.int32),
            pltpu.VMEM((_CBL, _HW), jnp.float32),
            pltpu.VMEM((_CBL, _BW), jnp.int32),
            pltpu.VMEM((_CBL, _BW), jnp.float32),
            pltpu.VMEM((_CBL, _HW), jnp.int32),
            pltpu.VMEM((_CBL, _HW), jnp.float32),
            pltpu.VMEM((_LANES,), jnp.float32),
            pltpu.SemaphoreType.DMA, pltpu.SemaphoreType.DMA,
            pltpu.SemaphoreType.DMA, pltpu.SemaphoreType.DMA,
            pltpu.SemaphoreType.DMA, pltpu.SemaphoreType.DMA,
            pltpu.SemaphoreType.DMA, pltpu.SemaphoreType.DMA,
        ],
    )
    def sc_kernel(pred_hbm, ain0, aout0, ain1, aout1, ain2, aout2,
                  dummy_b, dummy_h, out_hbm,
                  spmem, bidx0, bval0, hidx0, hval0,
                  bidx1, bval1, hidx1, hval1, stage,
                  ssb0, ssh0, ssb1, ssh1, sgb0, sgh0, sgb1, sgh1):
        cid = lax.axis_index("c")
        sid = lax.axis_index("s")
        wid = sid * 2 + cid

        @pl.when(sid == 0)
        def _stage_table():
            pltpu.sync_copy(pred_hbm, spmem)

        plsc.subcore_barrier()

        tq0 = jnp.where(wid < 8, 16 * wid, 128 + 15 * (wid - 8))
        nq = jnp.where(wid < 8, 16, 15)
        tq1 = tq0 + nq
        base_q = jnp.minimum(tq0, _NOCT - _NCH * (_CBL // 8))

        rules = ((ain0, aout0), (ain1, aout1), (ain2, aout2))
        bufs = ((bidx0, bval0, hidx0, hval0, ssb0, ssh0, sgb0, sgh0),
                (bidx1, bval1, hidx1, hval1, ssb1, ssh1, sgb1, sgh1))
        sched = [(r, c) for r in range(_NRULE) for c in range(_NCH)]
        nsched = len(sched)

        def stage_i(i):
            r, c = sched[i]
            bidx, _, hidx, _, ssb, ssh, _, _ = bufs[i % 2]
            ain, aout = rules[r]
            b0 = (base_q + c * (_CBL // 8)) * 8
            hs = pltpu.async_copy(ain.at[pl.ds(b0, _CBL), :], bidx, ssb)
            hh = pltpu.async_copy(aout.at[pl.ds(b0, _CBL), :], hidx, ssh)
            return hs, hh

        def fire_i(i, handles):
            hs, hh = handles
            hs.wait()
            hh.wait()
            bidx, bval, hidx, hval, _, _, sgb, sgh = bufs[i % 2]

            def fire(bl, _):
                pltpu.async_copy(spmem.at[bidx.at[bl]], bval.at[bl], sgb)
                pltpu.async_copy(spmem.at[hidx.at[bl]], hval.at[bl], sgh)
                return 0

            lax.fori_loop(0, _CBL, fire, 0)

        def drain_i(i):
            _, bval, _, hval, _, _, sgb, sgh = bufs[i % 2]
            pltpu.make_async_copy(dummy_b, bval, sgb).wait()
            pltpu.make_async_copy(dummy_h, hval, sgh).wait()

        def compute_i(i, acc):
            r, c = sched[i]
            _, bval, _, hval, _, _, _, _ = bufs[i % 2]
            q_base = base_q + c * (_CBL // 8)

            def block_body(bl, a):
                q = q_base + lax.shift_right_logical(bl, 3)
                keep = jnp.logical_and(q >= tq0, q < tq1)
                f = jnp.where(keep, jnp.float32(1.0), jnp.float32(0.0))
                for k in range(_BLK // _LANES):
                    s = k * _LANES
                    t = bval[bl, pl.ds(s, _LANES)]
                    for j in range(1, _BODY_LEN):
                        t = t * bval[bl, pl.ds(j * _BLK + s, _LANES)]
                    for j in range(_HEAD_LEN):
                        t = t * (jnp.float32(1.0)
                                 - hval[bl, pl.ds(j * _BLK + s, _LANES)])
                    a = a + t * f
                return a

            return lax.fori_loop(0, _CBL, block_body, acc)

        handles = stage_i(0)
        fire_i(0, handles)
        handles = stage_i(1)
        acc = jnp.zeros((_LANES,), jnp.float32)
        for i in range(nsched):
            drain_i(i)
            if i + 2 < nsched:
                next_handles = stage_i(i + 2)
            if i + 1 < nsched:
                fire_i(i + 1, handles)
                handles = next_handles if i + 2 < nsched else None
            acc = compute_i(i, acc)
            r, c = sched[i]
            if c == _NCH - 1:
                stage[...] = acc
                pltpu.sync_copy(stage, out_hbm.at[r, wid])
                acc = jnp.zeros((_LANES,), jnp.float32)

    return sc_kernel


_SC_KERNEL = _make_sc_kernel()


def _to_blocks(x, k):
    return (x[:_NBLK * _BLK].reshape(_NBLK, _BLK, k).transpose(0, 2, 1)
            .reshape(_NBLK, k * _BLK))


def kernel(predictions, rule_weights, A_in_0, A_out_0, A_in_1, A_out_1,
           A_in_2, A_out_2):
    ains = [_to_blocks(a, _BODY_LEN) for a in (A_in_0, A_in_1, A_in_2)]
    aouts = [_to_blocks(a, _HEAD_LEN) for a in (A_out_0, A_out_1, A_out_2)]
    dummy_b = jnp.zeros((_CBL, _BW), jnp.float32)
    dummy_h = jnp.zeros((_CBL, _HW), jnp.float32)
    partials = _SC_KERNEL(predictions, ains[0], aouts[0], ains[1], aouts[1],
                          ains[2], aouts[2], dummy_b, dummy_h)
    s = partials.sum(axis=(1, 2))

    tail = []
    for a_in, a_out in ((A_in_0, A_out_0), (A_in_1, A_out_1),
                        (A_in_2, A_out_2)):
        tb = jnp.prod(jnp.take(predictions, a_in[_NBLK * _BLK:], axis=0),
                      axis=-1)
        th = jnp.prod(1.0 - jnp.take(predictions, a_out[_NBLK * _BLK:],
                                     axis=0), axis=-1)
        tail.append(jnp.sum(tb * th))
    s = s + jnp.stack(tail)

    return jnp.sum(rule_weights * s) / jnp.float32(_N_GROUND)
